# baseline (speedup 1.0000x reference)
import dataclasses

import jax
import jax.numpy as jnp
from jax import lax
from jax.experimental import pallas as pl
from jax.experimental.pallas import tpu as pltpu
from jax.experimental.pallas import tpu_sc as plsc

_NC = 2
_NS = 16
_NW = _NC * _NS
_L = 16

_N = 256 * 32 * 32
_PER_W = _N // _NW


def _router_body(e_hbm, t_hbm, out_hbm, e_v, o_v, t_v, sem):
    wid = lax.axis_index("s") * _NC + lax.axis_index("c")
    base = wid * _PER_W
    cp = pltpu.async_copy(e_hbm.at[pl.ds(base, _PER_W)], e_v, sem)
    pltpu.sync_copy(t_hbm, t_v)
    tv = t_v[...]
    even = lax.iota(jnp.int32, _L) * 2
    ones = jnp.full((_L,), 1, jnp.int32)
    zeros = jnp.zeros((_L,), jnp.int32)
    cp.wait()

    @pl.loop(0, _PER_W, step=_L)
    def _(i):
        e = e_v[pl.ds(i, _L)]
        coarse = jnp.where(e <= tv, ones, zeros)
        fine = ones - coarse
        idx = even + 2 * i
        plsc.store_scatter(o_v, [idx], coarse)
        plsc.store_scatter(o_v, [idx + 1], fine)

    pltpu.sync_copy(o_v, out_hbm.at[pl.ds(2 * base, 2 * _PER_W)])


def kernel(entropy, threshold):
    flat = entropy.reshape(_N)
    tvec = jnp.full((_L,), threshold, jnp.float32)
    mesh = plsc.VectorSubcoreMesh(core_axis_name="c", subcore_axis_name="s")
    cp = pltpu.CompilerParams()
    if "needs_layout_passes" in pltpu.CompilerParams.__dataclass_fields__:
        cp = dataclasses.replace(cp, needs_layout_passes=False)
    run = pl.kernel(
        _router_body,
        out_type=jax.ShapeDtypeStruct((2 * _N,), jnp.int32),
        mesh=mesh,
        scratch_types=[
            pltpu.VMEM((_PER_W,), jnp.float32),
            pltpu.VMEM((2 * _PER_W,), jnp.int32),
            pltpu.VMEM((_L,), jnp.float32),
            pltpu.SemaphoreType.DMA,
        ],
        compiler_params=cp,
    )
    out = run(flat, tvec)
    return out.reshape(256, 32, 32, 2)


# speedup vs baseline: 10.1405x; 10.1405x over previous
import dataclasses

import jax
import jax.numpy as jnp
from jax import lax
from jax.experimental import pallas as pl
from jax.experimental.pallas import tpu as pltpu
from jax.experimental.pallas import tpu_sc as plsc

_NC = 2
_NS = 16
_NW = _NC * _NS
_L = 16

_H = 32
_W = 32
_B = 256


def _router_body(e_hbm, t_hbm, out_hbm, e_v, o_v, t_v, sem):
    h = lax.axis_index("s") * _NC + lax.axis_index("c")
    cp = pltpu.async_copy(e_hbm.at[h], e_v, sem)
    pltpu.sync_copy(t_hbm, t_v)
    tv = t_v[...]
    ones = jnp.full((_L,), 1, jnp.int32)
    zeros = jnp.zeros((_L,), jnp.int32)
    cp.wait()

    @pl.loop(0, _W)
    def _(w):
        @pl.loop(0, _B, step=_L)
        def _(b):
            e = e_v[w, pl.ds(b, _L)]
            coarse = jnp.where(e <= tv, ones, zeros)
            o_v[w, 0, pl.ds(b, _L)] = coarse
            o_v[w, 1, pl.ds(b, _L)] = ones - coarse

    pltpu.sync_copy(o_v, out_hbm.at[h])


def kernel(entropy, threshold):
    e_t = jnp.transpose(entropy, (1, 2, 0))
    tvec = jnp.full((_L,), threshold, jnp.float32)
    mesh = plsc.VectorSubcoreMesh(core_axis_name="c", subcore_axis_name="s")
    cp = pltpu.CompilerParams()
    if "needs_layout_passes" in pltpu.CompilerParams.__dataclass_fields__:
        cp = dataclasses.replace(cp, needs_layout_passes=False)
    run = pl.kernel(
        _router_body,
        out_type=jax.ShapeDtypeStruct((_H, _W, 2, _B), jnp.int32),
        mesh=mesh,
        scratch_types=[
            pltpu.VMEM((_W, _B), jnp.float32),
            pltpu.VMEM((_W, 2, _B), jnp.int32),
            pltpu.VMEM((_L,), jnp.float32),
            pltpu.SemaphoreType.DMA,
        ],
        compiler_params=cp,
    )
    out = run(e_t, tvec)
    return jnp.transpose(out, (3, 0, 1, 2))
